# Initial kernel scaffold; baseline (speedup 1.0000x reference)
import jax
import jax.numpy as jnp
from jax.experimental import pallas as pl


def kernel(feats, node_batches):
    raise NotImplementedError("write your pallas kernel here")


# speedup vs baseline: 1.1310x; 1.1310x over previous
import functools

import jax
import jax.numpy as jnp
from jax import lax
from jax.experimental import pallas as pl
from jax.experimental.pallas import tpu as pltpu
from jax.experimental.pallas import tpu_sc as plsc

N_GRAPHS = 1024
NODES_PER_GRAPH = 128
D_FEAT = 128
LANES = 16
NC, NS = 2, 16
NW = NC * NS
GPW = N_GRAPHS // NW
CH = D_FEAT // LANES
SCALE = 1.0 / NODES_PER_GRAPH


def _pool_body(feats_hbm, nb_hbm, out_hbm, idx_v, rows_v, out_v, sem):
    wid = lax.axis_index("s") * NC + lax.axis_index("c")
    base = wid * GPW
    pltpu.sync_copy(nb_hbm.at[pl.ds(base, GPW)], idx_v)
    for g in range(GPW):
        pltpu.async_copy(feats_hbm.at[idx_v.at[g]], rows_v, sem).wait()

        def body(r, accs):
            return tuple(accs[c] + rows_v[r, pl.ds(c * LANES, LANES)]
                         for c in range(CH))

        accs = lax.fori_loop(
            0, NODES_PER_GRAPH, body,
            tuple(jnp.zeros((LANES,), jnp.float32) for _ in range(CH)))
        for c in range(CH):
            out_v[g, pl.ds(c * LANES, LANES)] = accs[c] * SCALE
    pltpu.sync_copy(out_v, out_hbm.at[pl.ds(base, GPW)])


@jax.jit
def kernel(feats, node_batches):
    mesh = plsc.VectorSubcoreMesh(core_axis_name="c", subcore_axis_name="s")
    f = pl.kernel(
        _pool_body,
        mesh=mesh,
        out_type=jax.ShapeDtypeStruct((N_GRAPHS, D_FEAT), jnp.float32),
        scratch_types=[
            pltpu.VMEM((GPW, NODES_PER_GRAPH), jnp.int32),
            pltpu.VMEM((NODES_PER_GRAPH, D_FEAT), jnp.float32),
            pltpu.VMEM((GPW, D_FEAT), jnp.float32),
            pltpu.SemaphoreType.DMA,
        ],
    )
    return f(feats, node_batches)


# speedup vs baseline: 1.6734x; 1.4796x over previous
import functools

import jax
import jax.numpy as jnp
from jax import lax
from jax.experimental import pallas as pl
from jax.experimental.pallas import tpu as pltpu
from jax.experimental.pallas import tpu_sc as plsc

N_GRAPHS = 1024
NODES_PER_GRAPH = 128
D_FEAT = 128
LANES = 16
NC, NS = 2, 16
NW = NC * NS
GPW = N_GRAPHS // NW
CH = D_FEAT // LANES
SCALE = 1.0 / NODES_PER_GRAPH


def _pool_body(feats_hbm, nb_hbm, out_hbm, idx_v, rows_a, rows_b, out_v,
               sem_a, sem_b):
    wid = lax.axis_index("s") * NC + lax.axis_index("c")
    base = wid * GPW
    pltpu.sync_copy(nb_hbm.at[pl.ds(base, GPW)], idx_v)
    bufs = (rows_a, rows_b)
    sems = (sem_a, sem_b)
    copies = [None, None]
    copies[0] = pltpu.async_copy(feats_hbm.at[idx_v.at[0]], bufs[0], sems[0])
    for g in range(GPW):
        if g + 1 < GPW:
            copies[(g + 1) % 2] = pltpu.async_copy(
                feats_hbm.at[idx_v.at[g + 1]], bufs[(g + 1) % 2],
                sems[(g + 1) % 2])
        copies[g % 2].wait()
        rows_v = bufs[g % 2]

        def body(r, accs):
            return tuple(accs[c] + rows_v[r, pl.ds(c * LANES, LANES)]
                         for c in range(CH))

        accs = lax.fori_loop(
            0, NODES_PER_GRAPH, body,
            tuple(jnp.zeros((LANES,), jnp.float32) for _ in range(CH)),
            unroll=2)
        for c in range(CH):
            out_v[g, pl.ds(c * LANES, LANES)] = accs[c] * SCALE
    pltpu.sync_copy(out_v, out_hbm.at[pl.ds(base, GPW)])


@jax.jit
def kernel(feats, node_batches):
    mesh = plsc.VectorSubcoreMesh(core_axis_name="c", subcore_axis_name="s")
    f = pl.kernel(
        _pool_body,
        mesh=mesh,
        out_type=jax.ShapeDtypeStruct((N_GRAPHS, D_FEAT), jnp.float32),
        scratch_types=[
            pltpu.VMEM((GPW, NODES_PER_GRAPH), jnp.int32),
            pltpu.VMEM((NODES_PER_GRAPH, D_FEAT), jnp.float32),
            pltpu.VMEM((NODES_PER_GRAPH, D_FEAT), jnp.float32),
            pltpu.VMEM((GPW, D_FEAT), jnp.float32),
            pltpu.SemaphoreType.DMA,
            pltpu.SemaphoreType.DMA,
        ],
    )
    return f(feats, node_batches)
